# baseline (speedup 1.0000x reference)
import functools

import jax
import jax.numpy as jnp
from jax.experimental import pallas as pl

_HIGH = jax.lax.Precision.HIGHEST
_EPS_BN = 1e-5
_EPS_D = 1e-8

MBLK = 512


def _prep_body(feat_ref, w1_ref, g1_ref, b1_ref, tf_ref, w2_ref, g2_ref,
               b2_ref, f_ref, sc2_ref, sh2_ref):
    B = feat_ref.shape[0]
    N = feat_ref.shape[2]
    Mtot = tf_ref.shape[2]
    w1 = w1_ref[...]
    ys = [jnp.dot(w1, feat_ref[b], preferred_element_type=jnp.float32,
                  precision=_HIGH) for b in range(B)]
    cnt1 = float(B * N)
    mean1 = sum(jnp.sum(y, axis=1, keepdims=True) for y in ys) / cnt1
    var1 = sum(jnp.sum((y - mean1) ** 2, axis=1, keepdims=True)
               for y in ys) / cnt1
    sc1 = g1_ref[...] * jax.lax.rsqrt(var1 + _EPS_BN)
    sh1 = b1_ref[...] - mean1 * sc1
    for b in range(B):
        f_ref[b] = jnp.maximum(ys[b] * sc1 + sh1, 0.0)

    cnt2 = float(B * Mtot)
    w2 = w2_ref[...]
    xmean = sum(jnp.sum(tf_ref[b], axis=1, keepdims=True)
                for b in range(B)) / cnt2
    smom = sum(jax.lax.dot_general(tf_ref[b], tf_ref[b],
                                   (((1,), (1,)), ((), ())),
                                   preferred_element_type=jnp.float32,
                                   precision=_HIGH) for b in range(B))
    mu2 = jnp.dot(w2, xmean, preferred_element_type=jnp.float32,
                  precision=_HIGH)
    ey2 = jnp.sum(jnp.dot(w2, smom, preferred_element_type=jnp.float32,
                          precision=_HIGH) * w2, axis=1,
                  keepdims=True) / cnt2
    var2 = ey2 - mu2 * mu2
    sc2 = g2_ref[...] * jax.lax.rsqrt(var2 + _EPS_BN)
    sc2_ref[...] = sc2
    sh2_ref[...] = b2_ref[...] - mu2 * sc2


def _main_body(xyz_ref, txyz_ref, f_ref, tf_ref, w2_ref, sc2_ref, sh2_ref,
               o_ref):
    S = xyz_ref[0]
    T = txyz_ref[0]
    N = S.shape[0]
    MB = T.shape[1]
    d2 = None
    for c in range(3):
        diff = S[:, c:c + 1] - T[c:c + 1, :]
        d2 = diff * diff if d2 is None else d2 + diff * diff
    iota0 = jax.lax.broadcasted_iota(jnp.int32, (N, MB), 0)
    key = jax.lax.bitcast_convert_type(
        ((jax.lax.bitcast_convert_type(d2, jnp.int32) + jnp.int32(1024))
         & jnp.int32(~2047)) | iota0, jnp.float32)
    kmax = jnp.float32(jnp.inf)
    k0 = jnp.min(key, axis=0, keepdims=True)
    m1 = jnp.where(key == k0, kmax, key)
    k1 = jnp.min(m1, axis=0, keepdims=True)
    m2 = jnp.where(m1 == k1, kmax, m1)
    k2 = jnp.min(m2, axis=0, keepdims=True)
    recips = []
    for kk in (k0, k1, k2):
        dq = jax.lax.bitcast_convert_type(
            jax.lax.bitcast_convert_type(kk, jnp.int32) & jnp.int32(~2047),
            jnp.float32)
        recips.append(1.0 / (jnp.sqrt(dq) + _EPS_D))
    norm = recips[0] + recips[1] + recips[2]
    wmat = jnp.where(key == k0, recips[0] / norm,
                     jnp.where(key == k1, recips[1] / norm,
                               jnp.where(key == k2, recips[2] / norm, 0.0)))
    interp = jnp.dot(f_ref[0], wmat, preferred_element_type=jnp.float32,
                     precision=_HIGH)
    y2 = jnp.dot(w2_ref[...], tf_ref[0], preferred_element_type=jnp.float32,
                 precision=_HIGH)
    t = jnp.maximum(y2 * sc2_ref[...] + sh2_ref[...], 0.0)
    o_ref[0] = t + interp


@jax.jit
def kernel(xyz, feature, target_xyz, target_feature, W1, gamma1, beta1, W2,
           gamma2, beta2):
    B, N, _ = xyz.shape
    M = target_xyz.shape[1]
    C2 = W1.shape[0]
    txyz_t = jnp.transpose(target_xyz, (0, 2, 1))
    g1 = gamma1.reshape(C2, 1)
    b1 = beta1.reshape(C2, 1)
    g2 = gamma2.reshape(C2, 1)
    b2 = beta2.reshape(C2, 1)

    f, sc2, sh2 = pl.pallas_call(
        _prep_body,
        out_shape=[
            jax.ShapeDtypeStruct((B, C2, N), jnp.float32),
            jax.ShapeDtypeStruct((C2, 1), jnp.float32),
            jax.ShapeDtypeStruct((C2, 1), jnp.float32),
        ],
    )(feature, W1, g1, b1, target_feature, W2, g2, b2)

    grid = (B, M // MBLK)
    out = pl.pallas_call(
        _main_body,
        grid=grid,
        in_specs=[
            pl.BlockSpec((1, N, 3), lambda b, j: (b, 0, 0)),
            pl.BlockSpec((1, 3, MBLK), lambda b, j: (b, 0, j)),
            pl.BlockSpec((1, C2, N), lambda b, j: (b, 0, 0)),
            pl.BlockSpec((1, C2, MBLK), lambda b, j: (b, 0, j)),
            pl.BlockSpec((C2, C2), lambda b, j: (0, 0)),
            pl.BlockSpec((C2, 1), lambda b, j: (0, 0)),
            pl.BlockSpec((C2, 1), lambda b, j: (0, 0)),
        ],
        out_specs=pl.BlockSpec((1, C2, MBLK), lambda b, j: (b, 0, j)),
        out_shape=jax.ShapeDtypeStruct((B, C2, M), jnp.float32),
    )(xyz, txyz_t, f, target_feature, W2, sc2, sh2)
    return out


# speedup vs baseline: 1.0016x; 1.0016x over previous
import functools

import jax
import jax.numpy as jnp
from jax import lax
from jax.experimental import pallas as pl
from jax.experimental.pallas import tpu as pltpu
from jax.experimental.pallas import tpu_sc as plsc

_HIGH = jax.lax.Precision.HIGHEST
_EPS_BN = 1e-5
_EPS_D = 1e-8

MBLK = 512
CBLK = 2048
SC_CHUNK = 128


def _prep_body(featT_ref, w1t_ref, g1_ref, b1_ref, tf_ref, w2_ref, g2_ref,
               b2_ref, ft_ref, sc2_ref, sh2_ref):
    B = featT_ref.shape[0]
    N = featT_ref.shape[1]
    Mtot = tf_ref.shape[2]
    w1t = w1t_ref[...]
    ys = [jnp.dot(featT_ref[b], w1t, preferred_element_type=jnp.float32,
                  precision=_HIGH) for b in range(B)]
    cnt1 = float(B * N)
    mean1 = sum(jnp.sum(y, axis=0, keepdims=True) for y in ys) / cnt1
    var1 = sum(jnp.sum((y - mean1) ** 2, axis=0, keepdims=True)
               for y in ys) / cnt1
    sc1 = g1_ref[...] * jax.lax.rsqrt(var1 + _EPS_BN)
    sh1 = b1_ref[...] - mean1 * sc1
    for b in range(B):
        ft_ref[b] = jnp.maximum(ys[b] * sc1 + sh1, 0.0)

    cnt2 = float(B * Mtot)
    w2 = w2_ref[...]
    xmean = sum(jnp.sum(tf_ref[b], axis=1, keepdims=True)
                for b in range(B)) / cnt2
    smom = sum(jax.lax.dot_general(tf_ref[b], tf_ref[b],
                                   (((1,), (1,)), ((), ())),
                                   preferred_element_type=jnp.float32,
                                   precision=_HIGH) for b in range(B))
    mu2 = jnp.dot(w2, xmean, preferred_element_type=jnp.float32,
                  precision=_HIGH)
    ey2 = jnp.sum(jnp.dot(w2, smom, preferred_element_type=jnp.float32,
                          precision=_HIGH) * w2, axis=1,
                  keepdims=True) / cnt2
    var2 = ey2 - mu2 * mu2
    sc2 = g2_ref[...] * jax.lax.rsqrt(var2 + _EPS_BN)
    sc2_ref[...] = sc2
    sh2_ref[...] = b2_ref[...] - mu2 * sc2


def _main_body(xyz_ref, txyz_ref, tft_ref, w2t_ref, sc2_ref, sh2_ref,
               pt_ref, gi0_ref, gi1_ref, gi2_ref, w0_ref, w1_ref, w2o_ref):
    S = xyz_ref[0]
    T = txyz_ref[0]
    N = S.shape[0]
    d2 = None
    for c in range(3):
        diff = S[:, c:c + 1] - T[c:c + 1, :]
        d2 = diff * diff if d2 is None else d2 + diff * diff
    iota0 = jax.lax.broadcasted_iota(jnp.int32, d2.shape, 0)
    key = jax.lax.bitcast_convert_type(
        (jax.lax.bitcast_convert_type(d2, jnp.int32) & jnp.int32(~2047))
        | iota0, jnp.float32)
    kmax = jnp.float32(jnp.inf)
    k0 = jnp.min(key, axis=0, keepdims=True)
    m1 = jnp.where(key == k0, kmax, key)
    k1 = jnp.min(m1, axis=0, keepdims=True)
    m2 = jnp.where(m1 == k1, kmax, m1)
    k2 = jnp.min(m2, axis=0, keepdims=True)
    gbase = pl.program_id(0) * N
    recips = []
    for kk, gi_ref in ((k0, gi0_ref), (k1, gi1_ref), (k2, gi2_ref)):
        kbits = jax.lax.bitcast_convert_type(kk, jnp.int32)
        gi_ref[0] = (kbits & jnp.int32(2047)) + gbase
        dq = jax.lax.bitcast_convert_type(kbits & jnp.int32(~2047),
                                          jnp.float32)
        recips.append(1.0 / (jnp.sqrt(dq) + _EPS_D))
    norm = recips[0] + recips[1] + recips[2]
    w0_ref[0] = recips[0] / norm
    w1_ref[0] = recips[1] / norm
    w2o_ref[0] = recips[2] / norm
    y2 = jnp.dot(tft_ref[0], w2t_ref[...], preferred_element_type=jnp.float32,
                 precision=_HIGH)
    pt_ref[0] = jnp.maximum(y2 * sc2_ref[...] + sh2_ref[...], 0.0)


def _sc_gather_body(ftab, gi0, gi1, gi2, r0_out, r1_out, r2_out,
                    idx0, idx1, idx2, rb0, rb1, rb2, s0, s1, s2, rpw, nc):
    wid = lax.axis_index("s") * nc + lax.axis_index("c")
    base = wid * rpw
    for ch in range(rpw // SC_CHUNK):
        off = base + ch * SC_CHUNK
        pltpu.sync_copy(gi0.at[pl.ds(off, SC_CHUNK)], idx0)
        pltpu.sync_copy(gi1.at[pl.ds(off, SC_CHUNK)], idx1)
        pltpu.sync_copy(gi2.at[pl.ds(off, SC_CHUNK)], idx2)
        c0 = pltpu.async_copy(ftab.at[idx0], rb0, s0)
        c1 = pltpu.async_copy(ftab.at[idx1], rb1, s1)
        c2 = pltpu.async_copy(ftab.at[idx2], rb2, s2)
        c0.wait()
        c1.wait()
        c2.wait()
        pltpu.sync_copy(rb0, r0_out.at[pl.ds(off, SC_CHUNK)])
        pltpu.sync_copy(rb1, r1_out.at[pl.ds(off, SC_CHUNK)])
        pltpu.sync_copy(rb2, r2_out.at[pl.ds(off, SC_CHUNK)])


def _combine_body(pt_ref, r0_ref, r1_ref, r2_ref, w0_ref, w1_ref, w2_ref,
                  o_ref):
    o_ref[0] = (pt_ref[0]
                + w0_ref[0] * r0_ref[0]
                + w1_ref[0] * r1_ref[0]
                + w2_ref[0] * r2_ref[0])


@jax.jit
def kernel(xyz, feature, target_xyz, target_feature, W1, gamma1, beta1, W2,
           gamma2, beta2):
    B, N, _ = xyz.shape
    M = target_xyz.shape[1]
    C2 = W1.shape[0]
    BM = B * M
    txyz_t = jnp.transpose(target_xyz, (0, 2, 1))
    featT = jnp.transpose(feature, (0, 2, 1))
    tfT = jnp.transpose(target_feature, (0, 2, 1))
    g1 = gamma1.reshape(1, C2)
    b1 = beta1.reshape(1, C2)
    g2 = gamma2.reshape(C2, 1)
    b2 = beta2.reshape(C2, 1)

    ft, sc2, sh2 = pl.pallas_call(
        _prep_body,
        out_shape=[
            jax.ShapeDtypeStruct((B, N, C2), jnp.float32),
            jax.ShapeDtypeStruct((C2, 1), jnp.float32),
            jax.ShapeDtypeStruct((C2, 1), jnp.float32),
        ],
    )(featT, W1.T, g1, b1, target_feature, W2, g2, b2)

    grid = (B, M // MBLK)
    blk_row_i = jax.ShapeDtypeStruct((B, 1, M), jnp.int32)
    blk_row_f = jax.ShapeDtypeStruct((B, 1, M), jnp.float32)
    row_spec = pl.BlockSpec((1, 1, MBLK), lambda b, j: (b, 0, j))
    part, gi0, gi1, gi2, w0, w1, w2 = pl.pallas_call(
        _main_body,
        grid=grid,
        in_specs=[
            pl.BlockSpec((1, N, 3), lambda b, j: (b, 0, 0)),
            pl.BlockSpec((1, 3, MBLK), lambda b, j: (b, 0, j)),
            pl.BlockSpec((1, MBLK, C2), lambda b, j: (b, j, 0)),
            pl.BlockSpec((C2, C2), lambda b, j: (0, 0)),
            pl.BlockSpec((1, C2), lambda b, j: (0, 0)),
            pl.BlockSpec((1, C2), lambda b, j: (0, 0)),
        ],
        out_specs=[
            pl.BlockSpec((1, MBLK, C2), lambda b, j: (b, j, 0)),
            row_spec, row_spec, row_spec, row_spec, row_spec, row_spec,
        ],
        out_shape=[
            jax.ShapeDtypeStruct((B, M, C2), jnp.float32),
            blk_row_i, blk_row_i, blk_row_i,
            blk_row_f, blk_row_f, blk_row_f,
        ],
    )(xyz, txyz_t, tfT, W2.T, sc2.reshape(1, C2), sh2.reshape(1, C2))

    info = plsc.get_sparse_core_info()
    nw = info.num_cores * info.num_subcores
    rpw = BM // nw
    mesh = plsc.VectorSubcoreMesh(core_axis_name="c", subcore_axis_name="s")
    rows_ty = jax.ShapeDtypeStruct((BM, C2), jnp.float32)
    sc_fn = functools.partial(
        pl.kernel,
        mesh=mesh,
        out_type=[rows_ty, rows_ty, rows_ty],
        scratch_types=[
            pltpu.VMEM((SC_CHUNK,), jnp.int32),
            pltpu.VMEM((SC_CHUNK,), jnp.int32),
            pltpu.VMEM((SC_CHUNK,), jnp.int32),
            pltpu.VMEM((SC_CHUNK, C2), jnp.float32),
            pltpu.VMEM((SC_CHUNK, C2), jnp.float32),
            pltpu.VMEM((SC_CHUNK, C2), jnp.float32),
            pltpu.SemaphoreType.DMA,
            pltpu.SemaphoreType.DMA,
            pltpu.SemaphoreType.DMA,
        ],
    )(functools.partial(_sc_gather_body, rpw=rpw, nc=info.num_cores))
    r0, r1, r2 = sc_fn(ft.reshape(B * N, C2), gi0.reshape(BM),
                       gi1.reshape(BM), gi2.reshape(BM))

    cgrid = (B, M // CBLK)
    blk_spec = pl.BlockSpec((1, CBLK, C2), lambda b, j: (b, j, 0))
    col_spec = pl.BlockSpec((1, CBLK, 1), lambda b, j: (b, j, 0))
    out_t = pl.pallas_call(
        _combine_body,
        grid=cgrid,
        in_specs=[blk_spec, blk_spec, blk_spec, blk_spec,
                  col_spec, col_spec, col_spec],
        out_specs=blk_spec,
        out_shape=jax.ShapeDtypeStruct((B, M, C2), jnp.float32),
    )(part, r0.reshape(B, M, C2), r1.reshape(B, M, C2), r2.reshape(B, M, C2),
      jnp.transpose(w0, (0, 2, 1)), jnp.transpose(w1, (0, 2, 1)),
      jnp.transpose(w2, (0, 2, 1)))

    return jnp.transpose(out_t, (0, 2, 1))
